# baseline (speedup 1.0000x reference)
import functools

import jax
import jax.numpy as jnp
from jax import lax
from jax.experimental import pallas as pl
from jax.experimental.pallas import tpu as pltpu
from jax.experimental.pallas import tpu_sc as plsc

NC = 2
NS = 16
NW = NC * NS
C = 64
NCHUNK = 320
NCH0 = 240
NI = 8
NG = 1
OCT = 8


def _linear_body(x_ref, wt_ref, b_ref, o_ref):
    o_ref[...] = (
        jnp.dot(x_ref[...], wt_ref[...], preferred_element_type=jnp.float32)
        + b_ref[...]
    )


def _combine_body(p_ref, norm_ref, o_ref):
    n = o_ref.shape[0]
    o_ref[...] = norm_ref[...] * (p_ref[0, :n] + p_ref[1, :n])


@functools.lru_cache(maxsize=None)
def _make_agg(N, D):
    NP = ((N + 8 * NS) // (8 * NS)) * (8 * NS)
    ROWS = NP // NS

    mesh = plsc.VectorSubcoreMesh(
        core_axis_name="c", subcore_axis_name="s", num_cores=NC, num_subcores=NS
    )

    @functools.partial(
        pl.kernel,
        out_type=jax.ShapeDtypeStruct((NC, NP, D), jnp.float32),
        mesh=mesh,
        scratch_types=[
            pltpu.VMEM((NI, C), jnp.int32),
            pltpu.VMEM((NI, C), jnp.int32),
            pltpu.VMEM((NG, C, D), jnp.float32),
            pltpu.VMEM_SHARED((NP, D), jnp.float32),
            [pltpu.SemaphoreType.DMA] * NI,
            [pltpu.SemaphoreType.DMA] * NI,
            [pltpu.SemaphoreType.DMA] * NG,
        ],
    )
    def agg(h_hbm, src_hbm, dst_hbm, zeros_hbm, out_hbm,
            srcb_v, dstb_v, rows_v, acc_sh, semis, semid, semg):
        cid = lax.axis_index("c")
        sid = lax.axis_index("s")
        base = (sid * NCHUNK + cid * NCH0) * C
        pltpu.sync_copy(zeros_hbm, acc_sh.at[pl.ds(sid * ROWS, ROWS)])
        plsc.subcore_barrier()

        def idx_load(i, slot):
            off = base + i * C
            pltpu.async_copy(
                src_hbm.at[pl.ds(off, C)], srcb_v.at[slot], semis[slot])
            pltpu.async_copy(
                dst_hbm.at[pl.ds(off, C)], dstb_v.at[slot], semid[slot])

        def idx_wait(i, slot):
            off = base + i * C
            pltpu.make_async_copy(
                src_hbm.at[pl.ds(off, C)], srcb_v.at[slot], semis[slot]).wait()
            pltpu.make_async_copy(
                dst_hbm.at[pl.ds(off, C)], dstb_v.at[slot], semid[slot]).wait()

        def gather_start(islot, rslot):
            pltpu.async_copy(
                h_hbm.at[srcb_v.at[islot]], rows_v.at[rslot], semg[rslot])

        def gather_wait(islot, rslot):
            pltpu.make_async_copy(
                h_hbm.at[srcb_v.at[islot]], rows_v.at[rslot], semg[rslot]
            ).wait()

        def chunk_step(i, q, ahead, prefetch):
            idx_wait(i, q % NI)
            gather_start(q % NI, 0)
            gather_wait(q % NI, 0)
            pltpu.sync_copy(
                rows_v.at[0], acc_sh.at[dstb_v.at[q % NI]], add=True)
            if prefetch:
                idx_load(i + 7, (q + 7) % NI)

        def run_chunks(n_chunks):
            for j in range(NI - 1):
                idx_load(j, j)

            def oct_body(g, carry):
                for q in range(OCT):
                    chunk_step(g * OCT + q, q, True, True)
                return carry

            n_octs = n_chunks // OCT
            lax.fori_loop(0, n_octs - 1, oct_body, 0)
            i0 = (n_octs - 1) * OCT
            for q in range(OCT):
                i = i0 + q
                chunk_step(i, q, i + 3 < n_chunks, i + 7 < n_chunks)

        @pl.when(cid == 0)
        def _():
            run_chunks(NCH0)

        @pl.when(cid == 1)
        def _():
            run_chunks(NCHUNK - NCH0)

        plsc.subcore_barrier()
        pltpu.sync_copy(
            acc_sh.at[pl.ds(sid * ROWS, ROWS)],
            out_hbm.at[cid, pl.ds(sid * ROWS, ROWS)],
        )

    return agg


def kernel(x, edge_index, norm, W, b):
    N, D_in = x.shape
    D_out = W.shape[0]
    E = edge_index.shape[1]

    h = pl.pallas_call(
        _linear_body,
        out_shape=jax.ShapeDtypeStruct((N, D_out), jnp.float32),
    )(x, W.T, b.reshape(1, D_out))

    E_pad = NS * NCHUNK * C
    src = jnp.concatenate([edge_index[0], jnp.zeros((E_pad - E,), jnp.int32)])
    dst = jnp.concatenate([edge_index[1], jnp.full((E_pad - E,), N, jnp.int32)])
    NP = ((N + 8 * NS) // (8 * NS)) * (8 * NS)
    zeros = jnp.zeros((NP // NS, D_out), dtype=jnp.float32)
    partials = _make_agg(N, D_out)(h, src, dst, zeros)

    out = pl.pallas_call(
        _combine_body,
        out_shape=jax.ShapeDtypeStruct((N, D_out), jnp.float32),
    )(partials, norm)
    return out


# speedup vs baseline: 1.4875x; 1.4875x over previous
import functools

import jax
import jax.numpy as jnp
from jax import lax
from jax.experimental import pallas as pl
from jax.experimental.pallas import tpu as pltpu
from jax.experimental.pallas import tpu_sc as plsc

NC = 2
NS = 16
NW = NC * NS


def _linear_body(x_ref, wt_ref, b_ref, o_ref):
    o_ref[...] = (
        jnp.dot(x_ref[...], wt_ref[...], preferred_element_type=jnp.float32)
        + b_ref[...]
    )


def _combine_body(p_ref, norm_ref, o_ref):
    n = o_ref.shape[0]
    o_ref[...] = norm_ref[...] * (p_ref[0, :n] + p_ref[1, :n])


@functools.lru_cache(maxsize=None)
def _make_agg(N, D, E):
    EPW = E // NW
    C = 80
    n_chunks = EPW // C
    NP = ((N + 8 * NS - 1) // (8 * NS)) * (8 * NS)
    ROWS = NP // NS
    assert EPW * NW == E and n_chunks * C == EPW

    mesh = plsc.VectorSubcoreMesh(
        core_axis_name="c", subcore_axis_name="s", num_cores=NC, num_subcores=NS
    )

    @functools.partial(
        pl.kernel,
        out_type=jax.ShapeDtypeStruct((NC, NP, D), jnp.float32),
        mesh=mesh,
        scratch_types=[
            pltpu.VMEM((C,), jnp.int32),
            pltpu.VMEM((C,), jnp.int32),
            pltpu.VMEM((C, D), jnp.float32),
            pltpu.VMEM_SHARED((NP, D), jnp.float32),
            pltpu.SemaphoreType.DMA,
        ],
    )
    def agg(h_hbm, src_hbm, dst_hbm, zeros_hbm, out_hbm,
            src_v, dst_v, rows_v, acc_sh, sem):
        cid = lax.axis_index("c")
        sid = lax.axis_index("s")
        wid = sid * NC + cid
        pltpu.sync_copy(zeros_hbm, acc_sh.at[pl.ds(sid * ROWS, ROWS)])
        plsc.subcore_barrier()

        base = wid * EPW

        def body(i, carry):
            off = base + i * C
            pltpu.sync_copy(src_hbm.at[pl.ds(off, C)], src_v)
            pltpu.sync_copy(dst_hbm.at[pl.ds(off, C)], dst_v)
            pltpu.async_copy(h_hbm.at[src_v], rows_v, sem).wait()
            pltpu.sync_copy(rows_v, acc_sh.at[dst_v], add=True)
            return carry

        lax.fori_loop(0, n_chunks, body, 0)
        plsc.subcore_barrier()
        pltpu.sync_copy(
            acc_sh.at[pl.ds(sid * ROWS, ROWS)],
            out_hbm.at[cid, pl.ds(sid * ROWS, ROWS)],
        )

    return agg


def kernel(x, edge_index, norm, W, b):
    N, D_in = x.shape
    D_out = W.shape[0]
    E = edge_index.shape[1]

    h = pl.pallas_call(
        _linear_body,
        out_shape=jax.ShapeDtypeStruct((N, D_out), jnp.float32),
    )(x, W.T, b.reshape(1, D_out))

    src = edge_index[0]
    dst = edge_index[1]
    NP = ((N + 8 * NS - 1) // (8 * NS)) * (8 * NS)
    zeros = jnp.zeros((NP // NS, D_out), dtype=jnp.float32)
    partials = _make_agg(N, D_out, E)(h, src, dst, zeros)

    out = pl.pallas_call(
        _combine_body,
        out_shape=jax.ShapeDtypeStruct((N, D_out), jnp.float32),
    )(partials, norm)
    return out
